# Initial kernel scaffold; baseline (speedup 1.0000x reference)
import jax
import jax.numpy as jnp
from jax.experimental import pallas as pl


def kernel(x, ln_weight, ln_bias, base_weight, spline_weight):
    raise NotImplementedError("write your pallas kernel here")


# speedup vs baseline: 2.1772x; 2.1772x over previous
import functools
import math

import jax
import jax.numpy as jnp
from jax.experimental import pallas as pl
from jax.experimental.pallas import tpu as pltpu

_LN_EPS = 1e-5


def _kan_body(x_ref, lnw_ref, lnb_ref, w_ref, o_ref, a_ref, *, g, d_in,
              m_blk, r_chunk):
    n = pl.program_id(1)

    @pl.when(n == 0)
    def _():
        for r0 in range(0, m_blk, r_chunk):
            xv = x_ref[r0:r0 + r_chunk, :]
            mu = jnp.mean(xv, axis=1, keepdims=True)
            xc = xv - mu
            var = jnp.mean(xc * xc, axis=1, keepdims=True)
            xn = ((xc * jax.lax.rsqrt(var + _LN_EPS)) * lnw_ref[...]
                  + lnb_ref[...])
            a_ref[r0:r0 + r_chunk, 0:d_in] = (
                xn * jax.nn.sigmoid(xn)).astype(jnp.bfloat16)
            c1 = jnp.cos(jnp.float32(math.pi) * jnp.tanh(xn))
            a_ref[r0:r0 + r_chunk, d_in:2 * d_in] = c1.astype(jnp.bfloat16)
            two_c1 = 2.0 * c1
            ckm2, ckm1 = jnp.ones_like(c1), c1
            for k in range(2, g + 1):
                ck = two_c1 * ckm1 - ckm2
                a_ref[r0:r0 + r_chunk, k * d_in:(k + 1) * d_in] = (
                    ck.astype(jnp.bfloat16))
                ckm2, ckm1 = ckm1, ck

    o_ref[...] = jnp.dot(a_ref[...], w_ref[...],
                         preferred_element_type=jnp.float32)


def kernel(x, ln_weight, ln_bias, base_weight, spline_weight):
    b, s, d_in = x.shape
    d_out, _, g = spline_weight.shape
    m = b * s
    k = (g + 1) * d_in

    xm = x.reshape(m, d_in)
    w_cat = jnp.concatenate(
        [base_weight.T,
         jnp.transpose(spline_weight, (2, 1, 0)).reshape(g * d_in, d_out)],
        axis=0).astype(jnp.bfloat16)
    lnw = ln_weight.reshape(1, d_in)
    lnb = ln_bias.reshape(1, d_in)

    m_blk = min(512, m)
    n_blk = min(256, d_out)

    out = pl.pallas_call(
        functools.partial(_kan_body, g=g, d_in=d_in,
                          m_blk=m_blk, r_chunk=min(128, m_blk)),
        grid=(m // m_blk, d_out // n_blk),
        in_specs=[
            pl.BlockSpec((m_blk, d_in), lambda i, j: (i, 0)),
            pl.BlockSpec((1, d_in), lambda i, j: (0, 0)),
            pl.BlockSpec((1, d_in), lambda i, j: (0, 0)),
            pl.BlockSpec((k, n_blk), lambda i, j: (0, j)),
        ],
        out_specs=pl.BlockSpec((m_blk, n_blk), lambda i, j: (i, j)),
        out_shape=jax.ShapeDtypeStruct((m, d_out), jnp.float32),
        scratch_shapes=[pltpu.VMEM((m_blk, k), jnp.bfloat16)],
        compiler_params=pltpu.CompilerParams(
            dimension_semantics=("parallel", "arbitrary")),
        name="kan_fused",
    )(xm, lnw, lnb, w_cat)
    return out.reshape(b, s, d_out)


# speedup vs baseline: 2.5044x; 1.1503x over previous
import functools
import math

import jax
import jax.numpy as jnp
from jax.experimental import pallas as pl
from jax.experimental.pallas import tpu as pltpu

_LN_EPS = 1e-5

_COS_PI_COEFS = (
    1.000000000e+00, -4.934801579e+00, 4.058698177e+00, -1.335174441e+00,
    2.350634038e-01, -2.539114095e-02, 1.605373924e-03,
)


def _cos_pi(t):
    u = t * t
    acc = jnp.float32(_COS_PI_COEFS[-1])
    for a in _COS_PI_COEFS[-2::-1]:
        acc = acc * u + jnp.float32(a)
    return acc


def _kan_body(x_ref, lnw_ref, lnb_ref, w_ref, o_ref, a_ref, *, g, d_in,
              m_blk, r_chunk):
    n = pl.program_id(1)

    @pl.when(n == 0)
    def _():
        for r0 in range(0, m_blk, r_chunk):
            xv = x_ref[r0:r0 + r_chunk, :]
            mu = jnp.mean(xv, axis=1, keepdims=True)
            m2 = jnp.mean(xv * xv, axis=1, keepdims=True)
            var = m2 - mu * mu
            xn = (((xv - mu) * jax.lax.rsqrt(var + _LN_EPS)) * lnw_ref[...]
                  + lnb_ref[...])
            a_ref[r0:r0 + r_chunk, 0:d_in] = (
                xn * jax.nn.sigmoid(xn)).astype(jnp.bfloat16)
            c1 = _cos_pi(jnp.tanh(xn))
            a_ref[r0:r0 + r_chunk, d_in:2 * d_in] = c1.astype(jnp.bfloat16)
            two_c1 = 2.0 * c1
            ckm2, ckm1 = jnp.ones_like(c1), c1
            for k in range(2, g + 1):
                ck = two_c1 * ckm1 - ckm2
                a_ref[r0:r0 + r_chunk, k * d_in:(k + 1) * d_in] = (
                    ck.astype(jnp.bfloat16))
                ckm2, ckm1 = ckm1, ck

    o_ref[...] = jnp.dot(a_ref[...], w_ref[...],
                         preferred_element_type=jnp.float32)


def kernel(x, ln_weight, ln_bias, base_weight, spline_weight):
    b, s, d_in = x.shape
    d_out, _, g = spline_weight.shape
    m = b * s
    k = (g + 1) * d_in

    xm = x.reshape(m, d_in)
    w_cat = jnp.concatenate(
        [base_weight.T,
         jnp.transpose(spline_weight, (2, 1, 0)).reshape(g * d_in, d_out)],
        axis=0).astype(jnp.bfloat16)
    lnw = ln_weight.reshape(1, d_in)
    lnb = ln_bias.reshape(1, d_in)

    m_blk = min(512, m)
    n_blk = min(256, d_out)

    out = pl.pallas_call(
        functools.partial(_kan_body, g=g, d_in=d_in,
                          m_blk=m_blk, r_chunk=min(128, m_blk)),
        grid=(m // m_blk, d_out // n_blk),
        in_specs=[
            pl.BlockSpec((m_blk, d_in), lambda i, j: (i, 0)),
            pl.BlockSpec((1, d_in), lambda i, j: (0, 0)),
            pl.BlockSpec((1, d_in), lambda i, j: (0, 0)),
            pl.BlockSpec((k, n_blk), lambda i, j: (0, j)),
        ],
        out_specs=pl.BlockSpec((m_blk, n_blk), lambda i, j: (i, j)),
        out_shape=jax.ShapeDtypeStruct((m, d_out), jnp.float32),
        scratch_shapes=[pltpu.VMEM((m_blk, k), jnp.bfloat16)],
        compiler_params=pltpu.CompilerParams(
            dimension_semantics=("parallel", "arbitrary")),
        name="kan_fused",
    )(xm, lnw, lnb, w_cat)
    return out.reshape(b, s, d_out)


# speedup vs baseline: 2.9077x; 1.1610x over previous
import functools
import math

import jax
import jax.numpy as jnp
from jax.experimental import pallas as pl
from jax.experimental.pallas import tpu as pltpu

_LN_EPS = 1e-5

_COS_PI_COEFS = (
    1.000000000e+00, -4.934801579e+00, 4.058698177e+00, -1.335174441e+00,
    2.350634038e-01, -2.539114095e-02, 1.605373924e-03,
)


def _cos_pi(t):
    u = t * t
    acc = jnp.float32(_COS_PI_COEFS[-1])
    for a in _COS_PI_COEFS[-2::-1]:
        acc = acc * u + jnp.float32(a)
    return acc


def _kan_body(x_ref, lnw_ref, lnb_ref, w_ref, o_ref, a_ref, *, g, d_in,
              m_blk, r_chunk):
    n = pl.program_id(1)

    @pl.when(n == 0)
    def _():
        for r0 in range(0, m_blk, r_chunk):
            xv = x_ref[r0:r0 + r_chunk, :]
            mu = jnp.mean(xv, axis=1, keepdims=True)
            m2 = jnp.mean(xv * xv, axis=1, keepdims=True)
            var = m2 - mu * mu
            xn = (((xv - mu) * jax.lax.rsqrt(var + _LN_EPS)) * lnw_ref[...]
                  + lnb_ref[...])
            a_ref[r0:r0 + r_chunk, 0:d_in] = (
                xn * jax.nn.sigmoid(xn)).astype(jnp.bfloat16)
            c1 = _cos_pi(jnp.tanh(xn))
            a_ref[r0:r0 + r_chunk, d_in:2 * d_in] = c1.astype(jnp.bfloat16)
            two_c1 = 2.0 * c1
            ckm2, ckm1 = jnp.ones_like(c1), c1
            for k in range(2, g + 1):
                ck = two_c1 * ckm1 - ckm2
                a_ref[r0:r0 + r_chunk, k * d_in:(k + 1) * d_in] = (
                    ck.astype(jnp.bfloat16))
                ckm2, ckm1 = ckm1, ck
            o_ref[r0:r0 + r_chunk, :] = jnp.dot(
                a_ref[r0:r0 + r_chunk, :], w_ref[...],
                preferred_element_type=jnp.float32)

    @pl.when(n != 0)
    def _():
        o_ref[...] = jnp.dot(a_ref[...], w_ref[...],
                             preferred_element_type=jnp.float32)


def kernel(x, ln_weight, ln_bias, base_weight, spline_weight):
    b, s, d_in = x.shape
    d_out, _, g = spline_weight.shape
    m = b * s
    k = (g + 1) * d_in

    xm = x.reshape(m, d_in)
    w_cat = jnp.concatenate(
        [base_weight.T,
         jnp.transpose(spline_weight, (2, 1, 0)).reshape(g * d_in, d_out)],
        axis=0).astype(jnp.bfloat16)
    lnw = ln_weight.reshape(1, d_in)
    lnb = ln_bias.reshape(1, d_in)

    m_blk = min(512, m)
    n_blk = min(256, d_out)

    out = pl.pallas_call(
        functools.partial(_kan_body, g=g, d_in=d_in,
                          m_blk=m_blk, r_chunk=min(128, m_blk)),
        grid=(m // m_blk, d_out // n_blk),
        in_specs=[
            pl.BlockSpec((m_blk, d_in), lambda i, j: (i, 0)),
            pl.BlockSpec((1, d_in), lambda i, j: (0, 0)),
            pl.BlockSpec((1, d_in), lambda i, j: (0, 0)),
            pl.BlockSpec((k, n_blk), lambda i, j: (0, j)),
        ],
        out_specs=pl.BlockSpec((m_blk, n_blk), lambda i, j: (i, j)),
        out_shape=jax.ShapeDtypeStruct((m, d_out), jnp.float32),
        scratch_shapes=[pltpu.VMEM((m_blk, k), jnp.bfloat16)],
        compiler_params=pltpu.CompilerParams(
            dimension_semantics=("parallel", "arbitrary")),
        name="kan_fused",
    )(xm, lnw, lnb, w_cat)
    return out.reshape(b, s, d_out)


# speedup vs baseline: 2.9829x; 1.0259x over previous
import functools

import jax
import jax.numpy as jnp
from jax.experimental import pallas as pl
from jax.experimental.pallas import tpu as pltpu

_LN_EPS = 1e-5

_COS_PI_COEFS = (
    1.000000000e+00, -4.934801579e+00, 4.058698177e+00, -1.335174441e+00,
    2.350634038e-01, -2.539114095e-02, 1.605373924e-03,
)


def _cos_pi(t):
    u = t * t
    acc = jnp.float32(_COS_PI_COEFS[-1])
    for a in _COS_PI_COEFS[-2::-1]:
        acc = acc * u + jnp.float32(a)
    return acc


def _kan_body(x_ref, lnw_ref, lnb_ref, wb_ref, ws_ref, o_ref, ab_ref, as_ref,
              *, g, d_in, m_blk, r_chunk):
    n = pl.program_id(1)

    def _tile_dot(rows):
        return (jnp.dot(ab_ref[rows, :], wb_ref[...],
                        preferred_element_type=jnp.float32)
                + jnp.dot(as_ref[rows, :], ws_ref[...],
                          preferred_element_type=jnp.float32))

    @pl.when(n == 0)
    def _():
        for r0 in range(0, m_blk, r_chunk):
            rows = slice(r0, r0 + r_chunk)
            xv = x_ref[rows, :]
            mu = jnp.mean(xv, axis=1, keepdims=True)
            m2 = jnp.mean(xv * xv, axis=1, keepdims=True)
            var = m2 - mu * mu
            xn = (((xv - mu) * jax.lax.rsqrt(var + _LN_EPS)) * lnw_ref[...]
                  + lnb_ref[...])
            ab_ref[rows, :] = (xn * jax.nn.sigmoid(xn)).astype(jnp.bfloat16)
            c1 = _cos_pi(jnp.tanh(xn))
            as_ref[rows, 0:d_in] = c1.astype(jnp.bfloat16)
            two_c1 = 2.0 * c1
            ckm2, ckm1 = jnp.ones_like(c1), c1
            for k in range(2, g + 1):
                ck = two_c1 * ckm1 - ckm2
                as_ref[rows, (k - 1) * d_in:k * d_in] = ck.astype(jnp.bfloat16)
                ckm2, ckm1 = ckm1, ck
            o_ref[rows, :] = _tile_dot(rows)

    @pl.when(n != 0)
    def _():
        o_ref[...] = _tile_dot(slice(None))


def kernel(x, ln_weight, ln_bias, base_weight, spline_weight):
    b, s, d_in = x.shape
    d_out, _, g = spline_weight.shape
    m = b * s

    xm = x.reshape(m, d_in)
    wb = base_weight.T.astype(jnp.bfloat16)
    ws = jnp.transpose(spline_weight, (2, 1, 0)).reshape(
        g * d_in, d_out).astype(jnp.bfloat16)
    lnw = ln_weight.reshape(1, d_in)
    lnb = ln_bias.reshape(1, d_in)

    m_blk = min(512, m)
    n_blk = min(256, d_out)

    out = pl.pallas_call(
        functools.partial(_kan_body, g=g, d_in=d_in,
                          m_blk=m_blk, r_chunk=min(128, m_blk)),
        grid=(m // m_blk, d_out // n_blk),
        in_specs=[
            pl.BlockSpec((m_blk, d_in), lambda i, j: (i, 0)),
            pl.BlockSpec((1, d_in), lambda i, j: (0, 0)),
            pl.BlockSpec((1, d_in), lambda i, j: (0, 0)),
            pl.BlockSpec((d_in, n_blk), lambda i, j: (0, j)),
            pl.BlockSpec((g * d_in, n_blk), lambda i, j: (0, j)),
        ],
        out_specs=pl.BlockSpec((m_blk, n_blk), lambda i, j: (i, j)),
        out_shape=jax.ShapeDtypeStruct((m, d_out), jnp.float32),
        scratch_shapes=[pltpu.VMEM((m_blk, d_in), jnp.bfloat16),
                        pltpu.VMEM((m_blk, g * d_in), jnp.bfloat16)],
        compiler_params=pltpu.CompilerParams(
            dimension_semantics=("parallel", "arbitrary")),
        name="kan_fused",
    )(xm, lnw, lnb, wb, ws)
    return out.reshape(b, s, d_out)
